# baseline (speedup 1.0000x reference)
import jax
import jax.numpy as jnp
from jax.experimental import pallas as pl
from jax.experimental.pallas import tpu as pltpu

IMG = 224
P = 16
C = 384
NB = 4
KNN = 16
OUT = 1000
B = 32
HP = IMG // P
N = HP * HP
PDIM = 3 * P * P
POS = 3.0e38


def _dotf(a, b):
    return jax.lax.dot_general(a, b, (((1,), (0,)), ((), ())),
                               preferred_element_type=jnp.float32)


def _block_kernel(f_ref, sq_ref, sqt_ref, mr_ref):
    F = f_ref[0]
    G = jax.lax.dot_general(F, F, (((1,), (1,)), ((), ())),
                            preferred_element_type=jnp.float32)
    D = (sq_ref[0] + sqt_ref[0]) - 2.0 * G
    F1 = F.astype(jnp.bfloat16).astype(jnp.float32)
    r1 = F - F1
    F2 = r1.astype(jnp.bfloat16).astype(jnp.float32)
    F3 = r1 - F2
    iota_j = jax.lax.broadcasted_iota(jnp.int32, (N, N), 1)
    Dcur = D
    sels = []
    for _ in range(KNN):
        v = jnp.min(Dcur, axis=1, keepdims=True)
        jsel = jnp.min(jnp.where(Dcur <= v, iota_j, N), axis=1,
                       keepdims=True)
        sel = iota_j == jsel
        sels.append(sel.astype(jnp.float32))
        Dcur = jnp.where(sel, POS, Dcur)
    S = jnp.concatenate(sels, axis=0)
    g = (_dotf(S, F1) + _dotf(S, F2)) + _dotf(S, F3)
    M = jnp.max(g.reshape(KNN, N, C), axis=0)
    mr_ref[0] = M - F


def _head_kernel(h_ref, w1_ref, w2_ref, b2_ref, o_ref, acc_ref):
    k = pl.program_id(0)

    @pl.when(k == 0)
    def _init():
        acc_ref[...] = jnp.zeros_like(acc_ref)

    acc_ref[...] += jnp.dot(h_ref[...], w1_ref[...],
                            preferred_element_type=jnp.float32)

    @pl.when(k == pl.num_programs(0) - 1)
    def _fin():
        hm = jax.nn.gelu(acc_ref[...])
        o_ref[...] = jnp.dot(hm, w2_ref[...],
                             preferred_element_type=jnp.float32) + b2_ref[...]


def _maxrel(f):
    sq = jnp.sum(f * f, axis=-1)
    return pl.pallas_call(
        _block_kernel,
        grid=(B,),
        in_specs=[pl.BlockSpec((1, N, C), lambda b: (b, 0, 0)),
                  pl.BlockSpec((1, N, 1), lambda b: (b, 0, 0)),
                  pl.BlockSpec((1, 1, N), lambda b: (b, 0, 0))],
        out_specs=pl.BlockSpec((1, N, C), lambda b: (b, 0, 0)),
        out_shape=jax.ShapeDtypeStruct((B, N, C), jnp.float32),
        compiler_params=pltpu.CompilerParams(
            dimension_semantics=("parallel",)),
    )(f, sq[:, :, None], sq[:, None, :])


def kernel(x, patch_w, patch_b, block_w, block_b, bn_gamma, bn_beta,
           head_w1, head_w2, head_b2):
    y = jax.lax.conv_general_dilated(
        x, patch_w, window_strides=(P, P), padding='VALID',
        dimension_numbers=('NCHW', 'OIHW', 'NCHW')
    ) + patch_b[None, :, None, None]
    for i in range(NB):
        f = y.reshape(B, C, N).transpose(0, 2, 1)
        mr = _maxrel(f)
        z = jnp.concatenate([f, mr], axis=-1) @ block_w[i] + block_b[i]
        y = z.transpose(0, 2, 1).reshape(B, C, HP, HP)
        y = jax.nn.gelu(y)
        mean = jnp.mean(y, axis=(0, 2, 3), keepdims=True)
        var = jnp.var(y, axis=(0, 2, 3), keepdims=True)
        xh = (y - mean) / jnp.sqrt(var + 1e-5)
        y = (xh * bn_gamma[i][None, :, None, None]
             + bn_beta[i][None, :, None, None])
    f = y.reshape(B, C, N).transpose(0, 2, 1)

    hflat = f.transpose(0, 2, 1).reshape(B, C * N)
    kt = C * N // 12
    out = pl.pallas_call(
        _head_kernel,
        grid=(12,),
        in_specs=[pl.BlockSpec((B, kt), lambda k: (0, k)),
                  pl.BlockSpec((kt, 512), lambda k: (k, 0)),
                  pl.BlockSpec((512, OUT), lambda k: (0, 0)),
                  pl.BlockSpec((1, OUT), lambda k: (0, 0))],
        out_specs=pl.BlockSpec((B, OUT), lambda k: (0, 0)),
        out_shape=jax.ShapeDtypeStruct((B, OUT), jnp.float32),
        scratch_shapes=[pltpu.VMEM((B, 512), jnp.float32)],
    )(hflat, head_w1, head_w2, head_b2.reshape(1, OUT))
    return out


# speedup vs baseline: 1.2049x; 1.2049x over previous
import jax
import jax.numpy as jnp
from jax.experimental import pallas as pl
from jax.experimental.pallas import tpu as pltpu

IMG = 224
P = 16
C = 384
NB = 4
KNN = 16
OUT = 1000
B = 32
HP = IMG // P
N = HP * HP
PDIM = 3 * P * P
POS = 3.0e38


def _dotf(a, b):
    return jax.lax.dot_general(a, b, (((1,), (0,)), ((), ())),
                               preferred_element_type=jnp.float32)


def _block_kernel(f_ref, sq_ref, sqt_ref, mr_ref):
    F = f_ref[0]
    G = jax.lax.dot_general(F, F, (((1,), (1,)), ((), ())),
                            preferred_element_type=jnp.float32)
    D = (sq_ref[0] + sqt_ref[0]) - 2.0 * G
    F1 = F.astype(jnp.bfloat16).astype(jnp.float32)
    r1 = F - F1
    F2 = r1.astype(jnp.bfloat16).astype(jnp.float32)
    F3 = r1 - F2
    iota_j = jax.lax.broadcasted_iota(jnp.int32, (N, N), 1)
    Dcur = D
    M = None
    for _ in range(KNN):
        v = jnp.min(Dcur, axis=1, keepdims=True)
        jsel = jnp.min(jnp.where(Dcur <= v, iota_j, N), axis=1,
                       keepdims=True)
        sel = iota_j == jsel
        self32 = sel.astype(jnp.float32)
        g = (_dotf(self32, F1) + _dotf(self32, F2)) + _dotf(self32, F3)
        M = g if M is None else jnp.maximum(M, g)
        Dcur = jnp.where(sel, POS, Dcur)
    mr_ref[0] = M - F


def _head_kernel(h_ref, w1_ref, w2_ref, b2_ref, o_ref, acc_ref):
    k = pl.program_id(0)

    @pl.when(k == 0)
    def _init():
        acc_ref[...] = jnp.zeros_like(acc_ref)

    acc_ref[...] += jnp.dot(h_ref[...], w1_ref[...],
                            preferred_element_type=jnp.float32)

    @pl.when(k == pl.num_programs(0) - 1)
    def _fin():
        hm = jax.nn.gelu(acc_ref[...])
        o_ref[...] = jnp.dot(hm, w2_ref[...],
                             preferred_element_type=jnp.float32) + b2_ref[...]


def _maxrel(f):
    sq = jnp.sum(f * f, axis=-1)
    return pl.pallas_call(
        _block_kernel,
        grid=(B,),
        in_specs=[pl.BlockSpec((1, N, C), lambda b: (b, 0, 0)),
                  pl.BlockSpec((1, N, 1), lambda b: (b, 0, 0)),
                  pl.BlockSpec((1, 1, N), lambda b: (b, 0, 0))],
        out_specs=pl.BlockSpec((1, N, C), lambda b: (b, 0, 0)),
        out_shape=jax.ShapeDtypeStruct((B, N, C), jnp.float32),
        compiler_params=pltpu.CompilerParams(
            dimension_semantics=("parallel",)),
    )(f, sq[:, :, None], sq[:, None, :])


def kernel(x, patch_w, patch_b, block_w, block_b, bn_gamma, bn_beta,
           head_w1, head_w2, head_b2):
    y = jax.lax.conv_general_dilated(
        x, patch_w, window_strides=(P, P), padding='VALID',
        dimension_numbers=('NCHW', 'OIHW', 'NCHW')
    ) + patch_b[None, :, None, None]
    for i in range(NB):
        f = y.reshape(B, C, N).transpose(0, 2, 1)
        mr = _maxrel(f)
        z = jnp.concatenate([f, mr], axis=-1) @ block_w[i] + block_b[i]
        y = z.transpose(0, 2, 1).reshape(B, C, HP, HP)
        y = jax.nn.gelu(y)
        mean = jnp.mean(y, axis=(0, 2, 3), keepdims=True)
        var = jnp.var(y, axis=(0, 2, 3), keepdims=True)
        xh = (y - mean) / jnp.sqrt(var + 1e-5)
        y = (xh * bn_gamma[i][None, :, None, None]
             + bn_beta[i][None, :, None, None])
    f = y.reshape(B, C, N).transpose(0, 2, 1)

    hflat = f.transpose(0, 2, 1).reshape(B, C * N)
    kt = C * N // 12
    out = pl.pallas_call(
        _head_kernel,
        grid=(12,),
        in_specs=[pl.BlockSpec((B, kt), lambda k: (0, k)),
                  pl.BlockSpec((kt, 512), lambda k: (k, 0)),
                  pl.BlockSpec((512, OUT), lambda k: (0, 0)),
                  pl.BlockSpec((1, OUT), lambda k: (0, 0))],
        out_specs=pl.BlockSpec((B, OUT), lambda k: (0, 0)),
        out_shape=jax.ShapeDtypeStruct((B, OUT), jnp.float32),
        scratch_shapes=[pltpu.VMEM((B, 512), jnp.float32)],
    )(hflat, head_w1, head_w2, head_b2.reshape(1, OUT))
    return out
